# baseline (speedup 1.0000x reference)
import functools

import jax
import jax.numpy as jnp
from jax import lax
from jax.experimental import pallas as pl
from jax.experimental.pallas import tpu as pltpu
from jax.experimental.pallas import tpu_sc as plsc

VOCAB = 100000
WINDOW = 5000
BSZ = 16
BEAM = 8
K = 8
HALF = BEAM // 2
LANES = 16
NVEC = WINDOW // LANES
BLK = 8
BLKW = BLK * LANES
NBLK = NVEC // BLK
STEPS = 4
NEG_INF = float("-inf")
I32_BIG = 2**31 - 1


def _insert_topk(R, RI, x, xi):
    for lvl in range(K):
        swap = x > R[lvl]
        R[lvl], x = jnp.where(swap, x, R[lvl]), jnp.where(swap, R[lvl], x)
        RI[lvl], xi = jnp.where(swap, xi, RI[lvl]), jnp.where(swap, RI[lvl], xi)
    return R, RI


def _sc_body(step_hbm, orig_hbm, scores_hbm, lprobs_hbm, out_s, out_t, out_b,
             step_v, orig_v, scores_v, lbuf, blkmax_v, cand_v, cand_i,
             rowf, rowt, rowb, sh_v, sh_i, sem):
    c = lax.axis_index("c")
    s = lax.axis_index("s")
    batch = c * 8 + s // 2
    half = s % 2

    pltpu.sync_copy(step_hbm, step_v)
    pltpu.sync_copy(orig_hbm, orig_v)
    pltpu.sync_copy(scores_hbm, scores_v)
    iota = lax.iota(jnp.int32, LANES)
    neg = jnp.full((LANES,), NEG_INF, jnp.float32)

    bsplat = jnp.full((LANES,), batch, jnp.int32)
    ob = plsc.load_gather(orig_v, [bsplat])
    start_vec = (ob * 1000) % VOCAB
    start = pl.multiple_of(start_vec[0], 8)

    stepm1 = step_v[pl.ds(0, LANES)] - 1

    cps = []
    for j in range(HALF):
        jg = half * HALF + j
        cps.append(pltpu.async_copy(
            lprobs_hbm.at[batch, jg, pl.ds(start, WINDOW)],
            lbuf.at[pl.ds(j * WINDOW, WINDOW)], sem))
    for cp in cps:
        cp.wait()

    big = jnp.full((LANES,), I32_BIG, jnp.int32)

    def load_bias(j):
        bidx = (jnp.full((LANES,), batch * BEAM + half * HALF, jnp.int32) + j) \
            * STEPS + stepm1
        return plsc.load_gather(scores_v, [bidx])

    def beamA(j, M):
        bias = load_bias(j)
        joff = j * WINDOW

        def bodyA(b, M):
            off = joff + b * BLKW
            m = lbuf[pl.ds(off, LANES)]
            for u in range(1, BLK):
                m = jnp.maximum(m, lbuf[pl.ds(off + u * LANES, LANES)])
            m = m + bias
            blkmax_v[pl.ds((j * (NBLK + 1) + b) * LANES, LANES)] = m
            return jnp.maximum(M, m)

        M = lax.fori_loop(0, NBLK, bodyA, M)
        m = lbuf[pl.ds(joff + WINDOW - LANES, LANES)] + bias
        blkmax_v[pl.ds((j * (NBLK + 1) + NBLK) * LANES, LANES)] = m
        return jnp.maximum(M, m)

    M = lax.fori_loop(0, HALF, beamA, neg)

    ms, _ = plsc.sort_key_val(M, M)
    t = jnp.full((LANES,), ms[LANES - K])

    def beamB(j, car):
        bias = load_bias(j)
        base = (half * HALF + j) * VOCAB + start

        def bodyB(b, car):
            bm = blkmax_v[pl.ds((j * (NBLK + 1) + b) * LANES, LANES)]
            hit = plsc.all_reduce_population_count(bm >= t)[0] > 0

            def do(car):
                def vec(u, car):
                    off = jnp.where(b < NBLK, b * BLKW + u * LANES,
                                    WINDOW - LANES)
                    off = pl.multiple_of(off, 8)
                    x = lbuf[pl.ds(j * WINDOW + off, LANES)] + bias
                    xi = jnp.full((LANES,), base + off, jnp.int32) + iota
                    R, RI = _insert_topk(list(car[:K]), list(car[K:]), x, xi)
                    return tuple(R + RI)

                nv = jnp.where(b < NBLK, BLK, 1)
                return lax.fori_loop(0, nv, vec, car)

            return lax.cond(hit, do, lambda car: car, car)

        return lax.fori_loop(0, NBLK + 1, bodyB, car)

    carry = tuple([neg] * K + [jnp.zeros((LANES,), jnp.int32)] * K)
    carry = lax.fori_loop(0, HALF, beamB, carry)

    R = list(carry[:K])
    RI = list(carry[K:])
    for lvl in range(K):
        cand_v[pl.ds(lvl * LANES, LANES)] = R[lvl]
        cand_i[pl.ds(lvl * LANES, LANES)] = RI[lvl]

    pltpu.sync_copy(cand_v.at[pl.ds(0, K * LANES)], sh_v.at[s])
    pltpu.sync_copy(cand_i.at[pl.ds(0, K * LANES)], sh_i.at[s])
    plsc.subcore_barrier()

    @pl.when(half == 0)
    def _merge():
        pltpu.sync_copy(sh_v.at[s + 1], cand_v.at[pl.ds(K * LANES, K * LANES)])
        pltpu.sync_copy(sh_i.at[s + 1], cand_i.at[pl.ds(K * LANES, K * LANES)])

        def round_(r, acc):
            accv, acci = acc

            def mx(k2, m):
                return jnp.maximum(m, cand_v[pl.ds(k2 * LANES, LANES)])

            m = lax.fori_loop(0, 2 * K, mx, neg)
            gmax = jnp.full((LANES,), jnp.max(m))

            def mi(k2, mn):
                v = cand_v[pl.ds(k2 * LANES, LANES)]
                idr = cand_i[pl.ds(k2 * LANES, LANES)]
                return jnp.minimum(mn, jnp.where(v == gmax, idr, big))

            mn = lax.fori_loop(0, 2 * K, mi, big)
            gidx = jnp.full((LANES,), jnp.min(mn))

            def upd(k2, z):
                v = cand_v[pl.ds(k2 * LANES, LANES)]
                idr = cand_i[pl.ds(k2 * LANES, LANES)]
                cand_v[pl.ds(k2 * LANES, LANES)] = jnp.where(
                    (v == gmax) & (idr == gidx), neg, v)
                return z

            lax.fori_loop(0, 2 * K, upd, 0)
            accv = jnp.where(iota == r, gmax, accv)
            acci = jnp.where(iota == r, gidx, acci)
            return accv, acci

        accv, acci = lax.fori_loop(
            0, K, round_, (neg, jnp.zeros((LANES,), jnp.int32)))

        beams = jnp.zeros((LANES,), jnp.int32)
        for tt in range(1, BEAM):
            beams = beams + jnp.where(acci >= tt * VOCAB, 1, 0)
        toks = acci - beams * VOCAB
        rowf[...] = accv
        rowt[...] = toks
        rowb[...] = beams
        pltpu.sync_copy(rowf.at[pl.ds(0, K)], out_s.at[batch])
        pltpu.sync_copy(rowt.at[pl.ds(0, K)], out_t.at[batch])
        pltpu.sync_copy(rowb.at[pl.ds(0, K)], out_b.at[batch])


_sc_call = functools.partial(
    pl.kernel,
    out_type=[
        jax.ShapeDtypeStruct((BSZ, BEAM), jnp.float32),
        jax.ShapeDtypeStruct((BSZ, BEAM), jnp.int32),
        jax.ShapeDtypeStruct((BSZ, BEAM), jnp.int32),
    ],
    mesh=plsc.VectorSubcoreMesh(core_axis_name="c", subcore_axis_name="s"),
    scratch_types=[
        pltpu.VMEM((LANES,), jnp.int32),
        pltpu.VMEM((BSZ,), jnp.int32),
        pltpu.VMEM((BSZ * BEAM * STEPS,), jnp.float32),
        pltpu.VMEM((HALF * WINDOW,), jnp.float32),
        pltpu.VMEM((HALF * (NBLK + 1) * LANES,), jnp.float32),
        pltpu.VMEM((2 * K * LANES,), jnp.float32),
        pltpu.VMEM((2 * K * LANES,), jnp.int32),
        pltpu.VMEM((LANES,), jnp.float32),
        pltpu.VMEM((LANES,), jnp.int32),
        pltpu.VMEM((LANES,), jnp.int32),
        pltpu.VMEM_SHARED((LANES, K * LANES), jnp.float32),
        pltpu.VMEM_SHARED((LANES, K * LANES), jnp.int32),
        pltpu.SemaphoreType.DMA,
    ],
    compiler_params=pltpu.CompilerParams(
        use_tc_tiling_on_sc=False, needs_layout_passes=False),
)(_sc_body)


def kernel(step, lprobs, scores, prev_output_tokens, original_batch_idxs):
    step16 = jnp.broadcast_to(jnp.asarray(step, jnp.int32), (LANES,))
    scores_buf, indices_buf, beams_buf = _sc_call(
        step16, original_batch_idxs.astype(jnp.int32), scores.reshape(-1),
        lprobs)
    return scores_buf, indices_buf, beams_buf


# speedup vs baseline: 2.8114x; 2.8114x over previous
import functools

import jax
import jax.numpy as jnp
from jax import lax
from jax.experimental import pallas as pl
from jax.experimental import pallas as pl
from jax.experimental.pallas import tpu as pltpu
from jax.experimental.pallas import tpu_sc as plsc

VOCAB = 100000
WINDOW = 5000
BSZ = 16
BEAM = 8
K = 8
LANES = 16
STEPS = 4
TILE = 128
COVER = 2560
STRIDE = 2440
CL = 2688
NT = CL // TILE
BLK = TILE // LANES
NEG_INF = float("-inf")
I32_BIG = 2**31 - 1


def _insert_topk(R, RI, x, xi):
    for lvl in range(K):
        swap = x > R[lvl]
        R[lvl], x = jnp.where(swap, x, R[lvl]), jnp.where(swap, R[lvl], x)
        RI[lvl], xi = jnp.where(swap, xi, RI[lvl]), jnp.where(swap, RI[lvl], xi)
    return R, RI


def _sc_body(step_hbm, orig_hbm, scores_hbm, lprobs_hbm, out_s, out_t, out_b,
             step_v, orig_v, scores_v, lbuf, blkmax_v, cand_v, cand_i,
             rowf, rowt, rowb, stf, sti, omf, omt, omb,
             sh_v, sh_i, shoutf, shouti, sem):
    c = lax.axis_index("c")
    s = lax.axis_index("s")
    batch = c * 8 + s // 2
    half = s % 2

    pltpu.sync_copy(step_hbm, step_v)
    pltpu.sync_copy(orig_hbm, orig_v)
    pltpu.sync_copy(scores_hbm, scores_v)
    iota = lax.iota(jnp.int32, LANES)
    neg = jnp.full((LANES,), NEG_INF, jnp.float32)
    big = jnp.full((LANES,), I32_BIG, jnp.int32)

    bsplat = jnp.full((LANES,), batch, jnp.int32)
    ob = plsc.load_gather(orig_v, [bsplat])
    start_vec = (ob * 1000) % VOCAB
    start = start_vec[0]
    wbase = start + half * STRIDE
    c0 = pl.multiple_of((wbase // TILE) * TILE, TILE)
    delta = wbase - c0

    stepm1 = step_v[pl.ds(0, LANES)] - 1

    pltpu.async_copy(
        lprobs_hbm.at[batch, :, pl.ds(c0, CL)], lbuf, sem).wait()

    zlo = jnp.where(half == 0, 0, delta + COVER)
    zhi = jnp.where(half == 0, delta, CL)
    z16 = pl.multiple_of((zlo // LANES) * LANES, 8)
    nq = (zhi - z16 + LANES - 1) // LANES

    def clean_beam(j, z):
        def clean_q(q, z):
            rel = z16 + q * LANES
            rel = pl.multiple_of(rel, 8)
            v = lbuf[j, pl.ds(rel, LANES)]
            lane_rel = jnp.full((LANES,), rel, jnp.int32) + iota
            inz = (lane_rel >= zlo) & (lane_rel < zhi)
            lbuf[j, pl.ds(rel, LANES)] = jnp.where(inz, neg, v)
            return z

        return lax.fori_loop(0, nq, clean_q, z)

    lax.fori_loop(0, BEAM, clean_beam, 0)

    def load_bias(j):
        bidx = (jnp.full((LANES,), batch * BEAM, jnp.int32) + j) * STEPS + stepm1
        return plsc.load_gather(scores_v, [bidx])

    def beamA(j, M):
        bias = load_bias(j)

        def bodyA(b, M):
            off = b * TILE
            m = lbuf[j, pl.ds(off, LANES)]
            for u in range(1, BLK):
                m = jnp.maximum(m, lbuf[j, pl.ds(off + u * LANES, LANES)])
            m = m + bias
            blkmax_v[pl.ds((j * NT + b) * LANES, LANES)] = m
            return jnp.maximum(M, m)

        return lax.fori_loop(0, NT, bodyA, M)

    M = lax.fori_loop(0, BEAM, beamA, neg)

    ms, _ = plsc.sort_key_val(M, M)
    t = jnp.full((LANES,), ms[LANES - K])

    def beamB(j, car):
        bias = load_bias(j)
        base = j * VOCAB + c0

        def bodyB(b, car):
            bm = blkmax_v[pl.ds((j * NT + b) * LANES, LANES)]
            hit = plsc.all_reduce_population_count(bm >= t)[0] > 0

            def do(car):
                def vec(u, car):
                    off = b * TILE + u * LANES
                    x = lbuf[j, pl.ds(off, LANES)] + bias
                    xi = jnp.full((LANES,), base + off, jnp.int32) + iota
                    R, RI = _insert_topk(list(car[:K]), list(car[K:]), x, xi)
                    return tuple(R + RI)

                return lax.fori_loop(0, BLK, vec, car)

            return lax.cond(hit, do, lambda car: car, car)

        return lax.fori_loop(0, NT, bodyB, car)

    carry = tuple([neg] * K + [jnp.zeros((LANES,), jnp.int32)] * K)
    carry = lax.fori_loop(0, BEAM, beamB, carry)

    R = list(carry[:K])
    RI = list(carry[K:])
    for lvl in range(K):
        cand_v[pl.ds(lvl * LANES, LANES)] = R[lvl]
        cand_i[pl.ds(lvl * LANES, LANES)] = RI[lvl]

    pltpu.sync_copy(cand_v.at[pl.ds(0, K * LANES)], sh_v.at[pl.ds(s * K * LANES, K * LANES)])
    pltpu.sync_copy(cand_i.at[pl.ds(0, K * LANES)], sh_i.at[pl.ds(s * K * LANES, K * LANES)])
    plsc.subcore_barrier()

    @pl.when(half == 0)
    def _merge():
        pltpu.sync_copy(sh_v.at[pl.ds((s + 1) * K * LANES, K * LANES)],
                        cand_v.at[pl.ds(K * LANES, K * LANES)])
        pltpu.sync_copy(sh_i.at[pl.ds((s + 1) * K * LANES, K * LANES)],
                        cand_i.at[pl.ds(K * LANES, K * LANES)])

        def round_(r, acc):
            accv, acci = acc

            def mx(k2, m):
                return jnp.maximum(m, cand_v[pl.ds(k2 * LANES, LANES)])

            m = lax.fori_loop(0, 2 * K, mx, neg)
            gmax = jnp.full((LANES,), jnp.max(m))

            def mi(k2, mn):
                v = cand_v[pl.ds(k2 * LANES, LANES)]
                idr = cand_i[pl.ds(k2 * LANES, LANES)]
                return jnp.minimum(mn, jnp.where(v == gmax, idr, big))

            mn = lax.fori_loop(0, 2 * K, mi, big)
            gidx = jnp.full((LANES,), jnp.min(mn))

            def upd(k2, z):
                v = cand_v[pl.ds(k2 * LANES, LANES)]
                idr = cand_i[pl.ds(k2 * LANES, LANES)]
                cand_v[pl.ds(k2 * LANES, LANES)] = jnp.where(
                    (v == gmax) & (idr == gidx), neg, v)
                return z

            lax.fori_loop(0, 2 * K, upd, 0)
            accv = jnp.where(iota == r, gmax, accv)
            acci = jnp.where(iota == r, gidx, acci)
            return accv, acci

        accv, acci = lax.fori_loop(
            0, K, round_, (neg, jnp.zeros((LANES,), jnp.int32)))

        beams = jnp.zeros((LANES,), jnp.int32)
        for tt in range(1, BEAM):
            beams = beams + jnp.where(acci >= tt * VOCAB, 1, 0)
        toks = acci - beams * VOCAB
        rowf[...] = accv
        rowt[...] = toks
        rowb[...] = beams
        q = s // 2
        pltpu.sync_copy(rowf.at[pl.ds(0, K)], shoutf.at[pl.ds(q * K, K)])
        pltpu.sync_copy(rowt.at[pl.ds(0, K)], shouti.at[pl.ds(q * K, K)])
        pltpu.sync_copy(rowb.at[pl.ds(0, K)], shouti.at[pl.ds(64 + q * K, K)])

    plsc.subcore_barrier()

    @pl.when(s == 0)
    def _writeout():
        pltpu.sync_copy(shoutf, stf)
        pltpu.sync_copy(shouti, sti)
        for k in range(K):
            idx = jnp.minimum(jnp.full((LANES,), k * K, jnp.int32) + iota, 63)
            omf[k, pl.ds(0, LANES)] = plsc.load_gather(stf, [idx])
            omt[k, pl.ds(0, LANES)] = plsc.load_gather(sti, [idx])
            omb[k, pl.ds(0, LANES)] = plsc.load_gather(sti, [idx + 64])
        row0 = pl.multiple_of(c * 8, 8)
        pltpu.sync_copy(omf, out_s.at[pl.ds(row0, 8)])
        pltpu.sync_copy(omt, out_t.at[pl.ds(row0, 8)])
        pltpu.sync_copy(omb, out_b.at[pl.ds(row0, 8)])


_sc_call = functools.partial(
    pl.kernel,
    out_type=[
        jax.ShapeDtypeStruct((BSZ, TILE), jnp.float32),
        jax.ShapeDtypeStruct((BSZ, TILE), jnp.int32),
        jax.ShapeDtypeStruct((BSZ, TILE), jnp.int32),
    ],
    mesh=plsc.VectorSubcoreMesh(core_axis_name="c", subcore_axis_name="s"),
    scratch_types=[
        pltpu.VMEM((LANES,), jnp.int32),
        pltpu.VMEM((BSZ,), jnp.int32),
        pltpu.VMEM((BSZ * BEAM * STEPS,), jnp.float32),
        pltpu.VMEM((BEAM, CL), jnp.float32),
        pltpu.VMEM((BEAM * NT * LANES,), jnp.float32),
        pltpu.VMEM((2 * K * LANES,), jnp.float32),
        pltpu.VMEM((2 * K * LANES,), jnp.int32),
        pltpu.VMEM((LANES,), jnp.float32),
        pltpu.VMEM((LANES,), jnp.int32),
        pltpu.VMEM((LANES,), jnp.int32),
        pltpu.VMEM((64,), jnp.float32),
        pltpu.VMEM((128,), jnp.int32),
        pltpu.VMEM((K, TILE), jnp.float32),
        pltpu.VMEM((K, TILE), jnp.int32),
        pltpu.VMEM((K, TILE), jnp.int32),
        pltpu.VMEM_SHARED((LANES * K * LANES,), jnp.float32),
        pltpu.VMEM_SHARED((LANES * K * LANES,), jnp.int32),
        pltpu.VMEM_SHARED((64,), jnp.float32),
        pltpu.VMEM_SHARED((128,), jnp.int32),
        pltpu.SemaphoreType.DMA,
    ],
    compiler_params=pltpu.CompilerParams(needs_layout_passes=False),
)(_sc_body)


def kernel(step, lprobs, scores, prev_output_tokens, original_batch_idxs):
    step16 = jnp.broadcast_to(jnp.asarray(step, jnp.int32), (LANES,))
    o_s, o_t, o_b = _sc_call(
        step16, original_batch_idxs.astype(jnp.int32), scores.reshape(-1),
        lprobs)
    return o_s[:, :BEAM], o_t[:, :BEAM], o_b[:, :BEAM]


# speedup vs baseline: 2.8333x; 1.0078x over previous
import functools

import jax
import jax.numpy as jnp
from jax import lax
from jax.experimental import pallas as pl
from jax.experimental import pallas as pl
from jax.experimental.pallas import tpu as pltpu
from jax.experimental.pallas import tpu_sc as plsc

VOCAB = 100000
WINDOW = 5000
BSZ = 16
BEAM = 8
K = 8
LANES = 16
STEPS = 4
TILE = 128
COVER = 2560
STRIDE = 2440
CL = 2688
NT = CL // TILE
BLK = TILE // LANES
NEG_INF = float("-inf")
I32_BIG = 2**31 - 1


def _insert_topk(R, RI, x, xi):
    for lvl in range(K):
        swap = x > R[lvl]
        R[lvl], x = jnp.where(swap, x, R[lvl]), jnp.where(swap, R[lvl], x)
        RI[lvl], xi = jnp.where(swap, xi, RI[lvl]), jnp.where(swap, RI[lvl], xi)
    return R, RI


def _sc_body(step_hbm, orig_hbm, scores_hbm, lprobs_hbm, out_s, out_t, out_b,
             step_v, orig_v, scores_v, lbuf, blkmax_v, cand_v, cand_i,
             rowf, rowt, rowb, stf, sti, omf, omt, omb,
             sh_v, sh_i, shoutf, shouti, sem):
    c = lax.axis_index("c")
    s = lax.axis_index("s")
    batch = c * 8 + s // 2
    half = s % 2

    pltpu.sync_copy(step_hbm, step_v)
    pltpu.sync_copy(orig_hbm, orig_v)
    pltpu.sync_copy(scores_hbm, scores_v)
    iota = lax.iota(jnp.int32, LANES)
    neg = jnp.full((LANES,), NEG_INF, jnp.float32)
    big = jnp.full((LANES,), I32_BIG, jnp.int32)

    bsplat = jnp.full((LANES,), batch, jnp.int32)
    ob = plsc.load_gather(orig_v, [bsplat])
    start_vec = (ob * 1000) % VOCAB
    start = start_vec[0]
    wbase = start + half * STRIDE
    c0 = pl.multiple_of((wbase // TILE) * TILE, TILE)
    delta = wbase - c0

    stepm1 = step_v[pl.ds(0, LANES)] - 1

    pltpu.async_copy(
        lprobs_hbm.at[batch, :, pl.ds(c0, CL)], lbuf, sem).wait()

    zlo = jnp.where(half == 0, 0, delta + COVER)
    zhi = jnp.where(half == 0, delta, CL)
    z16 = pl.multiple_of((zlo // LANES) * LANES, 8)
    nq = (zhi - z16 + LANES - 1) // LANES

    def clean_beam(j, z):
        def clean_q(q, z):
            rel = z16 + q * LANES
            rel = pl.multiple_of(rel, 8)
            v = lbuf[j, pl.ds(rel, LANES)]
            lane_rel = jnp.full((LANES,), rel, jnp.int32) + iota
            inz = (lane_rel >= zlo) & (lane_rel < zhi)
            lbuf[j, pl.ds(rel, LANES)] = jnp.where(inz, neg, v)
            return z

        return lax.fori_loop(0, nq, clean_q, z)

    lax.fori_loop(0, BEAM, clean_beam, 0)

    def load_bias(j):
        bidx = (jnp.full((LANES,), batch * BEAM, jnp.int32) + j) * STEPS + stepm1
        return plsc.load_gather(scores_v, [bidx])

    def beamA(j, M):
        bias = load_bias(j)

        def bodyA(b, M):
            off = b * TILE
            m = lbuf[j, pl.ds(off, LANES)]
            for u in range(1, BLK):
                m = jnp.maximum(m, lbuf[j, pl.ds(off + u * LANES, LANES)])
            m = m + bias
            blkmax_v[pl.ds((j * NT + b) * LANES, LANES)] = m
            return jnp.maximum(M, m)

        return lax.fori_loop(0, NT, bodyA, M)

    M = lax.fori_loop(0, BEAM, beamA, neg)

    ms, _ = plsc.sort_key_val(M, M)
    t = jnp.full((LANES,), ms[LANES - K])

    def beamB(j, car):
        bias = load_bias(j)
        base = j * VOCAB + c0

        def bodyB(b, car):
            bm = blkmax_v[pl.ds((j * NT + b) * LANES, LANES)]
            hit = plsc.all_reduce_population_count(bm >= t)[0] > 0

            def do(car):
                xs, hits = [], []
                for u in range(BLK):
                    x = lbuf[j, pl.ds(b * TILE + u * LANES, LANES)] + bias
                    xs.append(x)
                    hits.append(plsc.all_reduce_population_count(x >= t)[0] > 0)
                for u in range(BLK):
                    def ins(car, _u=u):
                        off = b * TILE + _u * LANES
                        xi = jnp.full((LANES,), base + off, jnp.int32) + iota
                        R, RI = _insert_topk(list(car[:K]), list(car[K:]),
                                             xs[_u], xi)
                        return tuple(R + RI)

                    car = lax.cond(hits[u], ins, lambda car: car, car)
                return car

            return lax.cond(hit, do, lambda car: car, car)

        return lax.fori_loop(0, NT, bodyB, car)

    carry = tuple([neg] * K + [jnp.zeros((LANES,), jnp.int32)] * K)
    carry = lax.fori_loop(0, BEAM, beamB, carry)

    R = list(carry[:K])
    RI = list(carry[K:])
    for lvl in range(K):
        cand_v[pl.ds(lvl * LANES, LANES)] = R[lvl]
        cand_i[pl.ds(lvl * LANES, LANES)] = RI[lvl]

    pltpu.sync_copy(cand_v.at[pl.ds(0, K * LANES)], sh_v.at[pl.ds(s * K * LANES, K * LANES)])
    pltpu.sync_copy(cand_i.at[pl.ds(0, K * LANES)], sh_i.at[pl.ds(s * K * LANES, K * LANES)])
    plsc.subcore_barrier()

    @pl.when(half == 0)
    def _merge():
        pltpu.sync_copy(sh_v.at[pl.ds((s + 1) * K * LANES, K * LANES)],
                        cand_v.at[pl.ds(K * LANES, K * LANES)])
        pltpu.sync_copy(sh_i.at[pl.ds((s + 1) * K * LANES, K * LANES)],
                        cand_i.at[pl.ds(K * LANES, K * LANES)])

        vs0 = [cand_v[pl.ds(k2 * LANES, LANES)] for k2 in range(2 * K)]
        ids0 = [cand_i[pl.ds(k2 * LANES, LANES)] for k2 in range(2 * K)]

        def round_(r, acc):
            accv, acci, vs = acc[0], acc[1], list(acc[2])
            m = vs[0]
            for v in vs[1:]:
                m = jnp.maximum(m, v)
            gmax = jnp.full((LANES,), jnp.max(m))
            mn = big
            for k2 in range(2 * K):
                mn = jnp.minimum(mn, jnp.where(vs[k2] == gmax, ids0[k2], big))
            gidx = jnp.full((LANES,), jnp.min(mn))
            vs = [jnp.where((vs[k2] == gmax) & (ids0[k2] == gidx), neg, vs[k2])
                  for k2 in range(2 * K)]
            accv = jnp.where(iota == r, gmax, accv)
            acci = jnp.where(iota == r, gidx, acci)
            return (accv, acci, tuple(vs))

        accv, acci, _ = lax.fori_loop(
            0, K, round_,
            (neg, jnp.zeros((LANES,), jnp.int32), tuple(vs0)))

        beams = jnp.zeros((LANES,), jnp.int32)
        for tt in range(1, BEAM):
            beams = beams + jnp.where(acci >= tt * VOCAB, 1, 0)
        toks = acci - beams * VOCAB
        rowf[...] = accv
        rowt[...] = toks
        rowb[...] = beams
        q = s // 2
        pltpu.sync_copy(rowf.at[pl.ds(0, K)], shoutf.at[pl.ds(q * K, K)])
        pltpu.sync_copy(rowt.at[pl.ds(0, K)], shouti.at[pl.ds(q * K, K)])
        pltpu.sync_copy(rowb.at[pl.ds(0, K)], shouti.at[pl.ds(64 + q * K, K)])

    plsc.subcore_barrier()

    @pl.when(s == 0)
    def _writeout():
        pltpu.sync_copy(shoutf, stf)
        pltpu.sync_copy(shouti, sti)
        for k in range(K):
            idx = jnp.minimum(jnp.full((LANES,), k * K, jnp.int32) + iota, 63)
            omf[k, pl.ds(0, LANES)] = plsc.load_gather(stf, [idx])
            omt[k, pl.ds(0, LANES)] = plsc.load_gather(sti, [idx])
            omb[k, pl.ds(0, LANES)] = plsc.load_gather(sti, [idx + 64])
        row0 = pl.multiple_of(c * 8, 8)
        pltpu.sync_copy(omf, out_s.at[pl.ds(row0, 8)])
        pltpu.sync_copy(omt, out_t.at[pl.ds(row0, 8)])
        pltpu.sync_copy(omb, out_b.at[pl.ds(row0, 8)])


_sc_call = functools.partial(
    pl.kernel,
    out_type=[
        jax.ShapeDtypeStruct((BSZ, TILE), jnp.float32),
        jax.ShapeDtypeStruct((BSZ, TILE), jnp.int32),
        jax.ShapeDtypeStruct((BSZ, TILE), jnp.int32),
    ],
    mesh=plsc.VectorSubcoreMesh(core_axis_name="c", subcore_axis_name="s"),
    scratch_types=[
        pltpu.VMEM((LANES,), jnp.int32),
        pltpu.VMEM((BSZ,), jnp.int32),
        pltpu.VMEM((BSZ * BEAM * STEPS,), jnp.float32),
        pltpu.VMEM((BEAM, CL), jnp.float32),
        pltpu.VMEM((BEAM * NT * LANES,), jnp.float32),
        pltpu.VMEM((2 * K * LANES,), jnp.float32),
        pltpu.VMEM((2 * K * LANES,), jnp.int32),
        pltpu.VMEM((LANES,), jnp.float32),
        pltpu.VMEM((LANES,), jnp.int32),
        pltpu.VMEM((LANES,), jnp.int32),
        pltpu.VMEM((64,), jnp.float32),
        pltpu.VMEM((128,), jnp.int32),
        pltpu.VMEM((K, TILE), jnp.float32),
        pltpu.VMEM((K, TILE), jnp.int32),
        pltpu.VMEM((K, TILE), jnp.int32),
        pltpu.VMEM_SHARED((LANES * K * LANES,), jnp.float32),
        pltpu.VMEM_SHARED((LANES * K * LANES,), jnp.int32),
        pltpu.VMEM_SHARED((64,), jnp.float32),
        pltpu.VMEM_SHARED((128,), jnp.int32),
        pltpu.SemaphoreType.DMA,
    ],
    compiler_params=pltpu.CompilerParams(needs_layout_passes=False),
)(_sc_body)


def kernel(step, lprobs, scores, prev_output_tokens, original_batch_idxs):
    step16 = jnp.broadcast_to(jnp.asarray(step, jnp.int32), (LANES,))
    o_s, o_t, o_b = _sc_call(
        step16, original_batch_idxs.astype(jnp.int32), scores.reshape(-1),
        lprobs)
    return o_s[:, :BEAM], o_t[:, :BEAM], o_b[:, :BEAM]


# speedup vs baseline: 2.9732x; 1.0494x over previous
import functools

import jax
import jax.numpy as jnp
from jax import lax
from jax.experimental import pallas as pl
from jax.experimental import pallas as pl
from jax.experimental.pallas import tpu as pltpu
from jax.experimental.pallas import tpu_sc as plsc

VOCAB = 100000
WINDOW = 5000
BSZ = 16
BEAM = 8
K = 8
LANES = 16
STEPS = 4
TILE = 128
COVER = 2560
STRIDE = 2440
CL = 2688
NT = CL // TILE
GSZ = 7
BLK = TILE // LANES
NEG_INF = float("-inf")
I32_BIG = 2**31 - 1


def _insert_topk(R, RI, x, xi):
    for lvl in range(K):
        swap = x > R[lvl]
        R[lvl], x = jnp.where(swap, x, R[lvl]), jnp.where(swap, R[lvl], x)
        RI[lvl], xi = jnp.where(swap, xi, RI[lvl]), jnp.where(swap, RI[lvl], xi)
    return R, RI


def _sc_body(step_hbm, orig_hbm, scores_hbm, lprobs_hbm, out_s, out_t, out_b,
             step_v, orig_v, scores_v, lbuf, blkmax_v, cand_v, cand_i,
             rowf, rowt, rowb, stf, sti, omf, omt, omb,
             sh_v, sh_i, shoutf, shouti, sem):
    c = lax.axis_index("c")
    s = lax.axis_index("s")
    batch = c * 8 + s // 2
    half = s % 2

    pltpu.sync_copy(step_hbm, step_v)
    pltpu.sync_copy(orig_hbm, orig_v)
    pltpu.sync_copy(scores_hbm, scores_v)
    iota = lax.iota(jnp.int32, LANES)
    neg = jnp.full((LANES,), NEG_INF, jnp.float32)
    big = jnp.full((LANES,), I32_BIG, jnp.int32)

    bsplat = jnp.full((LANES,), batch, jnp.int32)
    ob = plsc.load_gather(orig_v, [bsplat])
    start_vec = (ob * 1000) % VOCAB
    start = start_vec[0]
    wbase = start + half * STRIDE
    c0 = pl.multiple_of((wbase // TILE) * TILE, TILE)
    delta = wbase - c0

    stepm1 = step_v[pl.ds(0, LANES)] - 1

    pltpu.async_copy(
        lprobs_hbm.at[batch, :, pl.ds(c0, CL)], lbuf, sem).wait()

    zlo = jnp.where(half == 0, 0, delta + COVER)
    zhi = jnp.where(half == 0, delta, CL)
    z16 = pl.multiple_of((zlo // LANES) * LANES, 8)
    nq = (zhi - z16 + LANES - 1) // LANES

    def clean_beam(j, z):
        def clean_q(q, z):
            rel = z16 + q * LANES
            rel = pl.multiple_of(rel, 8)
            v = lbuf[j, pl.ds(rel, LANES)]
            lane_rel = jnp.full((LANES,), rel, jnp.int32) + iota
            inz = (lane_rel >= zlo) & (lane_rel < zhi)
            lbuf[j, pl.ds(rel, LANES)] = jnp.where(inz, neg, v)
            return z

        return lax.fori_loop(0, nq, clean_q, z)

    lax.fori_loop(0, BEAM, clean_beam, 0)

    def load_bias(j):
        bidx = (jnp.full((LANES,), batch * BEAM, jnp.int32) + j) * STEPS + stepm1
        return plsc.load_gather(scores_v, [bidx])

    def beamA(j, M):
        bias = load_bias(j)

        def bodyA(b, M):
            off = b * TILE
            m = lbuf[j, pl.ds(off, LANES)]
            for u in range(1, BLK):
                m = jnp.maximum(m, lbuf[j, pl.ds(off + u * LANES, LANES)])
            m = m + bias
            blkmax_v[pl.ds((j * NT + b) * LANES, LANES)] = m
            return jnp.maximum(M, m)

        return lax.fori_loop(0, NT, bodyA, M)

    M = lax.fori_loop(0, BEAM, beamA, neg)

    ms, _ = plsc.sort_key_val(M, M)
    t = jnp.full((LANES,), ms[LANES - K])

    def beamB(j, car):
        bias = load_bias(j)
        base = j * VOCAB + c0

        def groupB(g, car):
            gm = blkmax_v[pl.ds((j * NT + g * GSZ) * LANES, LANES)]
            for v in range(1, GSZ):
                gm = jnp.maximum(
                    gm, blkmax_v[pl.ds((j * NT + g * GSZ + v) * LANES, LANES)])
            ghit = plsc.all_reduce_population_count(gm >= t)[0] > 0

            def dog(car):
                def tileB(v, car):
                    b = g * GSZ + v
                    bm = blkmax_v[pl.ds((j * NT + b) * LANES, LANES)]
                    hit = plsc.all_reduce_population_count(bm >= t)[0] > 0

                    def do(car):
                        def vec(u, car):
                            off = b * TILE + u * LANES
                            x = lbuf[j, pl.ds(off, LANES)] + bias
                            vhit = plsc.all_reduce_population_count(
                                x >= t)[0] > 0

                            def ins(car):
                                xi = jnp.full((LANES,), base + off,
                                              jnp.int32) + iota
                                R, RI = _insert_topk(
                                    list(car[:K]), list(car[K:]), x, xi)
                                return tuple(R + RI)

                            return lax.cond(vhit, ins, lambda car: car, car)

                        return lax.fori_loop(0, BLK, vec, car)

                    return lax.cond(hit, do, lambda car: car, car)

                return lax.fori_loop(0, GSZ, tileB, car)

            return lax.cond(ghit, dog, lambda car: car, car)

        return lax.fori_loop(0, NT // GSZ, groupB, car)

    carry = tuple([neg] * K + [jnp.zeros((LANES,), jnp.int32)] * K)
    carry = lax.fori_loop(0, BEAM, beamB, carry)

    R = list(carry[:K])
    RI = list(carry[K:])
    for lvl in range(K):
        cand_v[pl.ds(lvl * LANES, LANES)] = R[lvl]
        cand_i[pl.ds(lvl * LANES, LANES)] = RI[lvl]

    pltpu.sync_copy(cand_v.at[pl.ds(0, K * LANES)], sh_v.at[pl.ds(s * K * LANES, K * LANES)])
    pltpu.sync_copy(cand_i.at[pl.ds(0, K * LANES)], sh_i.at[pl.ds(s * K * LANES, K * LANES)])
    plsc.subcore_barrier()

    @pl.when(half == 0)
    def _merge():
        pltpu.sync_copy(sh_v.at[pl.ds((s + 1) * K * LANES, K * LANES)],
                        cand_v.at[pl.ds(K * LANES, K * LANES)])
        pltpu.sync_copy(sh_i.at[pl.ds((s + 1) * K * LANES, K * LANES)],
                        cand_i.at[pl.ds(K * LANES, K * LANES)])

        vs0 = [cand_v[pl.ds(k2 * LANES, LANES)] for k2 in range(2 * K)]
        ids0 = [cand_i[pl.ds(k2 * LANES, LANES)] for k2 in range(2 * K)]

        def round_(r, acc):
            accv, acci, vs = acc[0], acc[1], list(acc[2])
            m = vs[0]
            for v in vs[1:]:
                m = jnp.maximum(m, v)
            gmax = jnp.full((LANES,), jnp.max(m))
            mn = big
            for k2 in range(2 * K):
                mn = jnp.minimum(mn, jnp.where(vs[k2] == gmax, ids0[k2], big))
            gidx = jnp.full((LANES,), jnp.min(mn))
            vs = [jnp.where((vs[k2] == gmax) & (ids0[k2] == gidx), neg, vs[k2])
                  for k2 in range(2 * K)]
            accv = jnp.where(iota == r, gmax, accv)
            acci = jnp.where(iota == r, gidx, acci)
            return (accv, acci, tuple(vs))

        accv, acci, _ = lax.fori_loop(
            0, K, round_,
            (neg, jnp.zeros((LANES,), jnp.int32), tuple(vs0)))

        beams = jnp.zeros((LANES,), jnp.int32)
        for tt in range(1, BEAM):
            beams = beams + jnp.where(acci >= tt * VOCAB, 1, 0)
        toks = acci - beams * VOCAB
        rowf[...] = accv
        rowt[...] = toks
        rowb[...] = beams
        q = s // 2
        pltpu.sync_copy(rowf.at[pl.ds(0, K)], shoutf.at[pl.ds(q * K, K)])
        pltpu.sync_copy(rowt.at[pl.ds(0, K)], shouti.at[pl.ds(q * K, K)])
        pltpu.sync_copy(rowb.at[pl.ds(0, K)], shouti.at[pl.ds(64 + q * K, K)])

    plsc.subcore_barrier()

    @pl.when(s == 0)
    def _writeout():
        pltpu.sync_copy(shoutf, stf)
        pltpu.sync_copy(shouti, sti)
        for k in range(K):
            idx = jnp.minimum(jnp.full((LANES,), k * K, jnp.int32) + iota, 63)
            omf[k, pl.ds(0, LANES)] = plsc.load_gather(stf, [idx])
            omt[k, pl.ds(0, LANES)] = plsc.load_gather(sti, [idx])
            omb[k, pl.ds(0, LANES)] = plsc.load_gather(sti, [idx + 64])
        row0 = pl.multiple_of(c * 8, 8)
        pltpu.sync_copy(omf, out_s.at[pl.ds(row0, 8)])
        pltpu.sync_copy(omt, out_t.at[pl.ds(row0, 8)])
        pltpu.sync_copy(omb, out_b.at[pl.ds(row0, 8)])


_sc_call = functools.partial(
    pl.kernel,
    out_type=[
        jax.ShapeDtypeStruct((BSZ, TILE), jnp.float32),
        jax.ShapeDtypeStruct((BSZ, TILE), jnp.int32),
        jax.ShapeDtypeStruct((BSZ, TILE), jnp.int32),
    ],
    mesh=plsc.VectorSubcoreMesh(core_axis_name="c", subcore_axis_name="s"),
    scratch_types=[
        pltpu.VMEM((LANES,), jnp.int32),
        pltpu.VMEM((BSZ,), jnp.int32),
        pltpu.VMEM((BSZ * BEAM * STEPS,), jnp.float32),
        pltpu.VMEM((BEAM, CL), jnp.float32),
        pltpu.VMEM((BEAM * NT * LANES,), jnp.float32),
        pltpu.VMEM((2 * K * LANES,), jnp.float32),
        pltpu.VMEM((2 * K * LANES,), jnp.int32),
        pltpu.VMEM((LANES,), jnp.float32),
        pltpu.VMEM((LANES,), jnp.int32),
        pltpu.VMEM((LANES,), jnp.int32),
        pltpu.VMEM((64,), jnp.float32),
        pltpu.VMEM((128,), jnp.int32),
        pltpu.VMEM((K, TILE), jnp.float32),
        pltpu.VMEM((K, TILE), jnp.int32),
        pltpu.VMEM((K, TILE), jnp.int32),
        pltpu.VMEM_SHARED((LANES * K * LANES,), jnp.float32),
        pltpu.VMEM_SHARED((LANES * K * LANES,), jnp.int32),
        pltpu.VMEM_SHARED((64,), jnp.float32),
        pltpu.VMEM_SHARED((128,), jnp.int32),
        pltpu.SemaphoreType.DMA,
    ],
    compiler_params=pltpu.CompilerParams(needs_layout_passes=False),
)(_sc_body)


def kernel(step, lprobs, scores, prev_output_tokens, original_batch_idxs):
    step16 = jnp.broadcast_to(jnp.asarray(step, jnp.int32), (LANES,))
    o_s, o_t, o_b = _sc_call(
        step16, original_batch_idxs.astype(jnp.int32), scores.reshape(-1),
        lprobs)
    return o_s[:, :BEAM], o_t[:, :BEAM], o_b[:, :BEAM]
